# baseline (speedup 1.0000x reference)
import functools

import jax
import jax.numpy as jnp
from jax import lax
from jax.experimental import pallas as pl
from jax.experimental.pallas import tpu as pltpu
from jax.experimental.pallas import tpu_sc as plsc

N = 10000
E = 320000
F = 128
G = 64

NC, NS = 2, 16
NW = NC * NS
N_PAD = 10240
CHUNK = 128
IB = 8
NCHUNK0 = 160
E_PAD = NS * CHUNK * NCHUNK0
BLK = 512


def _seg_sum_sc(h, src0_t, dst0_t, zeros):
  mesh = plsc.VectorSubcoreMesh(
      core_axis_name="c", subcore_axis_name="s",
      num_cores=NC, num_subcores=NS)

  @functools.partial(
      pl.kernel,
      out_type=jax.ShapeDtypeStruct((N_PAD, F), jnp.float32),
      mesh=mesh,
      scratch_types=[
          pltpu.VMEM((3, IB, CHUNK), jnp.int32),
          pltpu.VMEM((3, IB, CHUNK), jnp.int32),
          pltpu.VMEM((2, CHUNK, F), jnp.float32),
          pltpu.VMEM_SHARED((N_PAD, F), jnp.float32),
          pltpu.SemaphoreType.DMA,
          pltpu.SemaphoreType.DMA,
          pltpu.SemaphoreType.DMA,
      ],
  )
  def k(h_hbm, src0_hbm, dst0_hbm, z_hbm, out_hbm,
        sidx, didx, rows, acc, gsem, isem, ssem):
    c = lax.axis_index("c")
    s = lax.axis_index("s")

    def pipeline(src_hbm, dst_hbm, nchunk):
      nb = nchunk // IB

      def block_descs(ib, slot):
        return (
            pltpu.make_async_copy(
                src_hbm.at[s, pl.ds(ib * IB, IB)], sidx.at[slot], isem),
            pltpu.make_async_copy(
                dst_hbm.at[s, pl.ds(ib * IB, IB)], didx.at[slot], isem),
        )

      def start_block(ib, slot):
        da, db = block_descs(ib, slot)
        da.start()
        db.start()

      def wait_block(ib, slot):
        da, db = block_descs(ib, slot)
        da.wait()
        db.wait()

      def gather(j, p):
        slot = lax.rem(j // IB, 3)
        jo = lax.rem(j, IB)
        return pltpu.make_async_copy(
            h_hbm.at[sidx.at[slot, jo]], rows.at[p], gsem)

      def scatter(j, p):
        slot = lax.rem(j // IB, 3)
        jo = lax.rem(j, IB)
        return pltpu.make_async_copy(
            rows.at[p], acc.at[didx.at[slot, jo]], ssem)

      start_block(0, 0)
      wait_block(0, 0)
      start_block(1, 1)
      gather(0, 0).start()

      def body(j, carry):
        p = lax.rem(j, 2)
        ib = j // IB
        jo = lax.rem(j, IB)
        gather(j, p).wait()

        @pl.when(j > 0)
        def _():
          scatter(j - 1, 1 - p).wait()

        @pl.when(jo == 0)
        def _():
          @pl.when(ib + 1 < nb)
          def _():
            wait_block(ib + 1, lax.rem(ib + 1, 3))

          @pl.when(ib + 2 < nb)
          def _():
            start_block(ib + 2, lax.rem(ib + 2, 3))

        @pl.when(j < nchunk - 1)
        def _():
          gather(j + 1, 1 - p).start()

        scatter(j, p).start(add=True)
        return carry

      lax.fori_loop(0, nchunk, body, 0)
      scatter(nchunk - 1, (nchunk - 1) % 2).wait()

    @pl.when(c == 0)
    def _():
      rows_per = N_PAD // NS
      pltpu.sync_copy(z_hbm.at[pl.ds(s * rows_per, rows_per)],
                      acc.at[pl.ds(s * rows_per, rows_per)])
      plsc.subcore_barrier()
      pipeline(src0_hbm, dst0_hbm, NCHUNK0)
      plsc.subcore_barrier()
      pltpu.sync_copy(acc.at[pl.ds(s * rows_per, rows_per)],
                      out_hbm.at[pl.ds(s * rows_per, rows_per)])

  return k(h, src0_t, dst0_t, zeros)


def _dot(a, b):
  return jnp.dot(a, b, preferred_element_type=jnp.float32)


def _conv_body(p_ref, h_ref, wrel_ref, wroot_ref, b_ref, o_ref):
  agg = p_ref[...]
  acc = _dot(agg, wrel_ref[...]) + _dot(h_ref[...], wroot_ref[...])
  o_ref[...] = jnp.maximum(acc + b_ref[...], 0.0)


def _conv1(p, h, wrel, wroot, b):
  dout = wrel.shape[1]
  grid = (N_PAD // BLK,)
  return pl.pallas_call(
      _conv_body,
      grid=grid,
      in_specs=[
          pl.BlockSpec((BLK, F), lambda i: (i, 0)),
          pl.BlockSpec((BLK, F), lambda i: (i, 0)),
          pl.BlockSpec(wrel.shape, lambda i: (0, 0)),
          pl.BlockSpec(wroot.shape, lambda i: (0, 0)),
          pl.BlockSpec(b.shape, lambda i: (0, 0)),
      ],
      out_specs=pl.BlockSpec((BLK, dout), lambda i: (i, 0)),
      out_shape=jax.ShapeDtypeStruct((N_PAD, dout), jnp.float32),
  )(p, h, wrel, wroot, b)


def _conv2_body(p_ref, h_ref, wrel_ref, wroot_ref, b_ref, oa_ref, ob_ref):
  agg = p_ref[...]
  acc = _dot(agg, wrel_ref[...]) + _dot(h_ref[...], wroot_ref[...])
  h2 = jnp.maximum(acc + b_ref[...], 0.0)
  oa_ref[...] = h2[:, :F]
  ob_ref[...] = h2[:, F:]


def _conv2(p, h, wrel, wroot, b):
  grid = (N_PAD // BLK,)
  return pl.pallas_call(
      _conv2_body,
      grid=grid,
      in_specs=[
          pl.BlockSpec((BLK, F), lambda i: (i, 0)),
          pl.BlockSpec((BLK, F), lambda i: (i, 0)),
          pl.BlockSpec(wrel.shape, lambda i: (0, 0)),
          pl.BlockSpec(wroot.shape, lambda i: (0, 0)),
          pl.BlockSpec(b.shape, lambda i: (0, 0)),
      ],
      out_specs=[
          pl.BlockSpec((BLK, F), lambda i: (i, 0)),
          pl.BlockSpec((BLK, F), lambda i: (i, 0)),
      ],
      out_shape=[
          jax.ShapeDtypeStruct((N_PAD, F), jnp.float32),
          jax.ShapeDtypeStruct((N_PAD, F), jnp.float32),
      ],
  )(p, h, wrel, wroot, b)


def _conv3_body(pa_ref, pb_ref, ha_ref, hb_ref, w3a_ref, w3b_ref, w3ra_ref,
                w3rb_ref, b_ref, w4rel_ref, o3_ref, o4_ref):
  agg_a = pa_ref[...]
  agg_b = pb_ref[...]
  acc = (_dot(agg_a, w3a_ref[...]) + _dot(agg_b, w3b_ref[...])
         + _dot(ha_ref[...], w3ra_ref[...]) + _dot(hb_ref[...], w3rb_ref[...]))
  h3 = jnp.maximum(acc + b_ref[...], 0.0)
  o3_ref[...] = h3
  o4_ref[...] = _dot(h3, w4rel_ref[...])


def _conv3(pa, pb, ha, hb, w3a, w3b, w3ra, w3rb, b, w4rel):
  grid = (N_PAD // BLK,)
  wspec = lambda w: pl.BlockSpec(w.shape, lambda i: (0, 0))
  return pl.pallas_call(
      _conv3_body,
      grid=grid,
      in_specs=[
          pl.BlockSpec((BLK, F), lambda i: (i, 0)),
          pl.BlockSpec((BLK, F), lambda i: (i, 0)),
          pl.BlockSpec((BLK, F), lambda i: (i, 0)),
          pl.BlockSpec((BLK, F), lambda i: (i, 0)),
          wspec(w3a), wspec(w3b), wspec(w3ra), wspec(w3rb), wspec(b),
          wspec(w4rel),
      ],
      out_specs=[
          pl.BlockSpec((BLK, 3 * F), lambda i: (i, 0)),
          pl.BlockSpec((BLK, F), lambda i: (i, 0)),
      ],
      out_shape=[
          jax.ShapeDtypeStruct((N_PAD, 3 * F), jnp.float32),
          jax.ShapeDtypeStruct((N_PAD, F), jnp.float32),
      ],
  )(pa, pb, ha, hb, w3a, w3b, w3ra, w3rb, b, w4rel)


def _pool_body(p_ref, h3_ref, batch_ref, w4root_ref, b4_ref, wh_ref, bh_ref,
               o_ref, sums, counts):
  i = pl.program_id(0)

  @pl.when(i == 0)
  def _():
    sums[...] = jnp.zeros_like(sums)
    counts[...] = jnp.zeros_like(counts)

  h4 = p_ref[...] + _dot(h3_ref[...], w4root_ref[...]) + b4_ref[...]
  seg = batch_ref[0]
  iota = lax.broadcasted_iota(jnp.int32, (G, BLK), 0)
  onehot_t = (iota == seg).astype(jnp.float32)
  sums[...] += _dot(onehot_t, h4)
  cnt = jnp.sum(onehot_t, axis=1, keepdims=True)
  counts[...] += jnp.broadcast_to(cnt, counts.shape)

  @pl.when(i == pl.num_programs(0) - 1)
  def _():
    pooled = sums[...] / jnp.maximum(counts[...], 1.0)
    o_ref[...] = _dot(pooled, wh_ref[...]) + bh_ref[...]


def _pool(p, h3, batch3, w4root, b4, wh_pad, bh_b):
  grid = (N_PAD // BLK,)
  wspec = lambda w: pl.BlockSpec(w.shape, lambda i: (0, 0))
  return pl.pallas_call(
      _pool_body,
      grid=grid,
      in_specs=[
          pl.BlockSpec((BLK, F), lambda i: (i, 0)),
          pl.BlockSpec((BLK, 3 * F), lambda i: (i, 0)),
          pl.BlockSpec((1, 1, BLK), lambda i: (i, 0, 0)),
          wspec(w4root), wspec(b4), wspec(wh_pad), wspec(bh_b),
      ],
      out_specs=pl.BlockSpec((G, F), lambda i: (0, 0)),
      out_shape=jax.ShapeDtypeStruct((G, F), jnp.float32),
      scratch_shapes=[
          pltpu.VMEM((G, F), jnp.float32),
          pltpu.VMEM((G, F), jnp.float32),
      ],
  )(p, h3, batch3, w4root, b4, wh_pad, bh_b)


def kernel(x, edge_index, batch, W1_rel, b1, W1_root, W2_rel, b2, W2_root,
           W3_rel, b3, W3_root, W4_rel, b4, W4_root, Wh, bh):
  src, dst = edge_index[0], edge_index[1]
  pad_e = E_PAD - E
  src_p = jnp.concatenate([src, jnp.zeros((pad_e,), jnp.int32)])
  pad_dst = (N + 8) + jnp.arange(pad_e, dtype=jnp.int32) % (N_PAD - N - 16)
  dst_p = jnp.concatenate([dst, pad_dst])
  src0_t = src_p.reshape(NS, NCHUNK0, CHUNK)
  dst0_t = dst_p.reshape(NS, NCHUNK0, CHUNK)
  zeros = jnp.zeros((N_PAD, F), jnp.float32)

  x_pad = jnp.pad(x, ((0, N_PAD - N), (0, 0)))
  batch3 = jnp.pad(batch, (0, N_PAD - N), constant_values=G).reshape(
      N_PAD // BLK, 1, BLK)

  b1r, b2r, b3r, b4r = (b.reshape(1, -1) for b in (b1, b2, b3, b4))
  w3a, w3b = W3_rel[:F], W3_rel[F:]
  w3ra, w3rb = W3_root[:F], W3_root[F:]
  wh_pad = jnp.pad(Wh, ((0, 0), (0, F - Wh.shape[1])))
  bh_b = jnp.broadcast_to(bh.reshape(1, -1), (1, F))

  p1 = _seg_sum_sc(x_pad, src0_t, dst0_t, zeros)
  h1 = _conv1(p1, x_pad, W1_rel, W1_root, b1r)
  p2 = _seg_sum_sc(h1, src0_t, dst0_t, zeros)
  h2a, h2b = _conv2(p2, h1, W2_rel, W2_root, b2r)
  p3a = _seg_sum_sc(h2a, src0_t, dst0_t, zeros)
  p3b = _seg_sum_sc(h2b, src0_t, dst0_t, zeros)
  h3, y4 = _conv3(p3a, p3b, h2a, h2b, w3a, w3b, w3ra, w3rb, b3r, W4_rel)
  p4 = _seg_sum_sc(y4, src0_t, dst0_t, zeros)
  out_full = _pool(p4, h3, batch3, W4_root, b4r, wh_pad, bh_b)
  return out_full[:, :1]


# speedup vs baseline: 3.8625x; 3.8625x over previous
import functools

import jax
import jax.numpy as jnp
from jax import lax
from jax.experimental import pallas as pl
from jax.experimental.pallas import tpu as pltpu
from jax.experimental.pallas import tpu_sc as plsc

N = 10000
E = 320000
F = 128
G = 64

NC, NS = 2, 16
NW = NC * NS
N_PAD = 10240
CHUNK = 128
IB = 8
NCHUNK0 = 80
NCHUNK1 = 80
E_PAD = NS * CHUNK * (NCHUNK0 + NCHUNK1)
E_SPLIT = NS * CHUNK * NCHUNK0
BLK = 512


def _seg_sum_sc(h, src0_t, dst0_t, src1_t, dst1_t, zeros):
  mesh = plsc.VectorSubcoreMesh(
      core_axis_name="c", subcore_axis_name="s",
      num_cores=NC, num_subcores=NS)

  @functools.partial(
      pl.kernel,
      out_type=jax.ShapeDtypeStruct((NC, N_PAD, F), jnp.float32),
      mesh=mesh,
      scratch_types=[
          pltpu.VMEM((3, IB, CHUNK), jnp.int32),
          pltpu.VMEM((3, IB, CHUNK), jnp.int32),
          pltpu.VMEM((2, CHUNK, F), jnp.float32),
          pltpu.VMEM_SHARED((N_PAD, F), jnp.float32),
          pltpu.SemaphoreType.DMA,
          pltpu.SemaphoreType.DMA,
          pltpu.SemaphoreType.DMA,
      ],
  )
  def k(h_hbm, src0_hbm, dst0_hbm, src1_hbm, dst1_hbm, z_hbm, out_hbm,
        sidx, didx, rows, acc, gsem, isem, ssem):
    c = lax.axis_index("c")
    s = lax.axis_index("s")

    def pipeline(src_hbm, dst_hbm, nchunk):
      nb = nchunk // IB

      def block_descs(ib, slot):
        return (
            pltpu.make_async_copy(
                src_hbm.at[s, pl.ds(ib * IB, IB)], sidx.at[slot], isem),
            pltpu.make_async_copy(
                dst_hbm.at[s, pl.ds(ib * IB, IB)], didx.at[slot], isem),
        )

      def start_block(ib, slot):
        da, db = block_descs(ib, slot)
        da.start()
        db.start()

      def wait_block(ib, slot):
        da, db = block_descs(ib, slot)
        da.wait()
        db.wait()

      def gather(j, p):
        slot = lax.rem(j // IB, 3)
        jo = lax.rem(j, IB)
        return pltpu.make_async_copy(
            h_hbm.at[sidx.at[slot, jo]], rows.at[p], gsem)

      def scatter(j, p):
        slot = lax.rem(j // IB, 3)
        jo = lax.rem(j, IB)
        return pltpu.make_async_copy(
            rows.at[p], acc.at[didx.at[slot, jo]], ssem)

      start_block(0, 0)
      wait_block(0, 0)
      start_block(1, 1)
      gather(0, 0).start()

      def body(j, carry):
        p = lax.rem(j, 2)
        ib = j // IB
        jo = lax.rem(j, IB)
        gather(j, p).wait()

        @pl.when(j > 0)
        def _():
          scatter(j - 1, 1 - p).wait()

        @pl.when(jo == 0)
        def _():
          @pl.when(ib + 1 < nb)
          def _():
            wait_block(ib + 1, lax.rem(ib + 1, 3))

          @pl.when(ib + 2 < nb)
          def _():
            start_block(ib + 2, lax.rem(ib + 2, 3))

        @pl.when(j < nchunk - 1)
        def _():
          gather(j + 1, 1 - p).start()

        scatter(j, p).start(add=True)
        return carry

      lax.fori_loop(0, nchunk, body, 0)
      scatter(nchunk - 1, (nchunk - 1) % 2).wait()

    rows_per = N_PAD // NS
    pltpu.sync_copy(z_hbm.at[pl.ds(s * rows_per, rows_per)],
                    acc.at[pl.ds(s * rows_per, rows_per)])
    plsc.subcore_barrier()

    @pl.when(c == 0)
    def _():
      pipeline(src0_hbm, dst0_hbm, NCHUNK0)

    @pl.when(c == 1)
    def _():
      pipeline(src1_hbm, dst1_hbm, NCHUNK1)

    plsc.subcore_barrier()
    pltpu.sync_copy(acc.at[pl.ds(s * rows_per, rows_per)],
                    out_hbm.at[c, pl.ds(s * rows_per, rows_per)])

  return k(h, src0_t, dst0_t, src1_t, dst1_t, zeros)


def _dot(a, b):
  return jnp.dot(a, b, preferred_element_type=jnp.float32)


def _row_mask():
  i = pl.program_id(0)
  rows = i * BLK + lax.broadcasted_iota(jnp.int32, (BLK, 1), 0)
  return (rows < N).astype(jnp.float32)


def _conv_body(p_ref, h_ref, wrel_ref, wroot_ref, b_ref, o_ref):
  agg = p_ref[0] + p_ref[1]
  acc = _dot(agg, wrel_ref[...]) + _dot(h_ref[...], wroot_ref[...])
  o_ref[...] = jnp.maximum(acc + b_ref[...], 0.0) * _row_mask()


def _conv1(p, h, wrel, wroot, b):
  dout = wrel.shape[1]
  grid = (N_PAD // BLK,)
  return pl.pallas_call(
      _conv_body,
      grid=grid,
      in_specs=[
          pl.BlockSpec((NC, BLK, F), lambda i: (0, i, 0)),
          pl.BlockSpec((BLK, F), lambda i: (i, 0)),
          pl.BlockSpec(wrel.shape, lambda i: (0, 0)),
          pl.BlockSpec(wroot.shape, lambda i: (0, 0)),
          pl.BlockSpec(b.shape, lambda i: (0, 0)),
      ],
      out_specs=pl.BlockSpec((BLK, dout), lambda i: (i, 0)),
      out_shape=jax.ShapeDtypeStruct((N_PAD, dout), jnp.float32),
  )(p, h, wrel, wroot, b)


def _conv2_body(p_ref, h_ref, wrel_ref, wroot_ref, b_ref, oa_ref, ob_ref):
  agg = p_ref[0] + p_ref[1]
  acc = _dot(agg, wrel_ref[...]) + _dot(h_ref[...], wroot_ref[...])
  h2 = jnp.maximum(acc + b_ref[...], 0.0) * _row_mask()
  oa_ref[...] = h2[:, :F]
  ob_ref[...] = h2[:, F:]


def _conv2(p, h, wrel, wroot, b):
  grid = (N_PAD // BLK,)
  return pl.pallas_call(
      _conv2_body,
      grid=grid,
      in_specs=[
          pl.BlockSpec((NC, BLK, F), lambda i: (0, i, 0)),
          pl.BlockSpec((BLK, F), lambda i: (i, 0)),
          pl.BlockSpec(wrel.shape, lambda i: (0, 0)),
          pl.BlockSpec(wroot.shape, lambda i: (0, 0)),
          pl.BlockSpec(b.shape, lambda i: (0, 0)),
      ],
      out_specs=[
          pl.BlockSpec((BLK, F), lambda i: (i, 0)),
          pl.BlockSpec((BLK, F), lambda i: (i, 0)),
      ],
      out_shape=[
          jax.ShapeDtypeStruct((N_PAD, F), jnp.float32),
          jax.ShapeDtypeStruct((N_PAD, F), jnp.float32),
      ],
  )(p, h, wrel, wroot, b)


def _conv3_body(pa_ref, pb_ref, ha_ref, hb_ref, w3a_ref, w3b_ref, w3ra_ref,
                w3rb_ref, b_ref, w4rel_ref, o3_ref, o4_ref):
  agg_a = pa_ref[0] + pa_ref[1]
  agg_b = pb_ref[0] + pb_ref[1]
  acc = (_dot(agg_a, w3a_ref[...]) + _dot(agg_b, w3b_ref[...])
         + _dot(ha_ref[...], w3ra_ref[...]) + _dot(hb_ref[...], w3rb_ref[...]))
  h3 = jnp.maximum(acc + b_ref[...], 0.0)
  o3_ref[...] = h3
  o4_ref[...] = _dot(h3, w4rel_ref[...]) * _row_mask()


def _conv3(pa, pb, ha, hb, w3a, w3b, w3ra, w3rb, b, w4rel):
  grid = (N_PAD // BLK,)
  wspec = lambda w: pl.BlockSpec(w.shape, lambda i: (0, 0))
  return pl.pallas_call(
      _conv3_body,
      grid=grid,
      in_specs=[
          pl.BlockSpec((NC, BLK, F), lambda i: (0, i, 0)),
          pl.BlockSpec((NC, BLK, F), lambda i: (0, i, 0)),
          pl.BlockSpec((BLK, F), lambda i: (i, 0)),
          pl.BlockSpec((BLK, F), lambda i: (i, 0)),
          wspec(w3a), wspec(w3b), wspec(w3ra), wspec(w3rb), wspec(b),
          wspec(w4rel),
      ],
      out_specs=[
          pl.BlockSpec((BLK, 3 * F), lambda i: (i, 0)),
          pl.BlockSpec((BLK, F), lambda i: (i, 0)),
      ],
      out_shape=[
          jax.ShapeDtypeStruct((N_PAD, 3 * F), jnp.float32),
          jax.ShapeDtypeStruct((N_PAD, F), jnp.float32),
      ],
  )(pa, pb, ha, hb, w3a, w3b, w3ra, w3rb, b, w4rel)


def _pool_body(p_ref, h3_ref, batch_ref, w4root_ref, b4_ref, wh_ref, bh_ref,
               o_ref, sums, counts):
  i = pl.program_id(0)

  @pl.when(i == 0)
  def _():
    sums[...] = jnp.zeros_like(sums)
    counts[...] = jnp.zeros_like(counts)

  h4 = (p_ref[0] + p_ref[1] + _dot(h3_ref[...], w4root_ref[...])
        + b4_ref[...])
  seg = batch_ref[0]
  iota = lax.broadcasted_iota(jnp.int32, (G, BLK), 0)
  onehot_t = (iota == seg).astype(jnp.float32)
  sums[...] += _dot(onehot_t, h4)
  cnt = jnp.sum(onehot_t, axis=1, keepdims=True)
  counts[...] += jnp.broadcast_to(cnt, counts.shape)

  @pl.when(i == pl.num_programs(0) - 1)
  def _():
    pooled = sums[...] / jnp.maximum(counts[...], 1.0)
    o_ref[...] = _dot(pooled, wh_ref[...]) + bh_ref[...]


def _pool(p, h3, batch3, w4root, b4, wh_pad, bh_b):
  grid = (N_PAD // BLK,)
  wspec = lambda w: pl.BlockSpec(w.shape, lambda i: (0, 0))
  return pl.pallas_call(
      _pool_body,
      grid=grid,
      in_specs=[
          pl.BlockSpec((NC, BLK, F), lambda i: (0, i, 0)),
          pl.BlockSpec((BLK, 3 * F), lambda i: (i, 0)),
          pl.BlockSpec((1, 1, BLK), lambda i: (i, 0, 0)),
          wspec(w4root), wspec(b4), wspec(wh_pad), wspec(bh_b),
      ],
      out_specs=pl.BlockSpec((G, F), lambda i: (0, 0)),
      out_shape=jax.ShapeDtypeStruct((G, F), jnp.float32),
      scratch_shapes=[
          pltpu.VMEM((G, F), jnp.float32),
          pltpu.VMEM((G, F), jnp.float32),
      ],
  )(p, h3, batch3, w4root, b4, wh_pad, bh_b)


def kernel(x, edge_index, batch, W1_rel, b1, W1_root, W2_rel, b2, W2_root,
           W3_rel, b3, W3_root, W4_rel, b4, W4_root, Wh, bh):
  src, dst = edge_index[0], edge_index[1]
  pad_e = E_PAD - E
  ar = jnp.arange(pad_e, dtype=jnp.int32)
  pad_src = N + 8 + ar % (N_PAD - N - 16)
  pad_dst = (ar * 17) % N_PAD
  src_p = jnp.concatenate([src, pad_src])
  dst_p = jnp.concatenate([dst, pad_dst])
  src0_t = src_p[:E_SPLIT].reshape(NS, NCHUNK0, CHUNK)
  dst0_t = dst_p[:E_SPLIT].reshape(NS, NCHUNK0, CHUNK)
  src1_t = src_p[E_SPLIT:].reshape(NS, NCHUNK1, CHUNK)
  dst1_t = dst_p[E_SPLIT:].reshape(NS, NCHUNK1, CHUNK)
  zeros = jnp.zeros((N_PAD, F), jnp.float32)

  x_pad = jnp.pad(x, ((0, N_PAD - N), (0, 0)))
  batch3 = jnp.pad(batch, (0, N_PAD - N), constant_values=G).reshape(
      N_PAD // BLK, 1, BLK)

  b1r, b2r, b3r, b4r = (b.reshape(1, -1) for b in (b1, b2, b3, b4))
  w3a, w3b = W3_rel[:F], W3_rel[F:]
  w3ra, w3rb = W3_root[:F], W3_root[F:]
  wh_pad = jnp.pad(Wh, ((0, 0), (0, F - Wh.shape[1])))
  bh_b = jnp.broadcast_to(bh.reshape(1, -1), (1, F))

  p1 = _seg_sum_sc(x_pad, src0_t, dst0_t, src1_t, dst1_t, zeros)
  h1 = _conv1(p1, x_pad, W1_rel, W1_root, b1r)
  p2 = _seg_sum_sc(h1, src0_t, dst0_t, src1_t, dst1_t, zeros)
  h2a, h2b = _conv2(p2, h1, W2_rel, W2_root, b2r)
  p3a = _seg_sum_sc(h2a, src0_t, dst0_t, src1_t, dst1_t, zeros)
  p3b = _seg_sum_sc(h2b, src0_t, dst0_t, src1_t, dst1_t, zeros)
  h3, y4 = _conv3(p3a, p3b, h2a, h2b, w3a, w3b, w3ra, w3rb, b3r, W4_rel)
  p4 = _seg_sum_sc(y4, src0_t, dst0_t, src1_t, dst1_t, zeros)
  out_full = _pool(p4, h3, batch3, W4_root, b4r, wh_pad, bh_b)
  return out_full[:, :1]


# speedup vs baseline: 3.9843x; 1.0315x over previous
import functools

import jax
import jax.numpy as jnp
from jax import lax
from jax.experimental import pallas as pl
from jax.experimental.pallas import tpu as pltpu
from jax.experimental.pallas import tpu_sc as plsc

N = 10000
E = 320000
F = 128
G = 64

NC, NS = 2, 16
NW = NC * NS
N_PAD = 10240
CHUNK = 128
IB = 8
NCHUNK0 = 80
NCHUNK1 = 80
E_PAD = NS * CHUNK * (NCHUNK0 + NCHUNK1)
E_SPLIT = NS * CHUNK * NCHUNK0
BLK = 1024


def _seg_sum_sc(h, src0_t, dst0_t, src1_t, dst1_t, zeros):
  mesh = plsc.VectorSubcoreMesh(
      core_axis_name="c", subcore_axis_name="s",
      num_cores=NC, num_subcores=NS)

  @functools.partial(
      pl.kernel,
      out_type=jax.ShapeDtypeStruct((NC, N_PAD, F), jnp.float32),
      mesh=mesh,
      scratch_types=[
          pltpu.VMEM((3, IB, CHUNK), jnp.int32),
          pltpu.VMEM((3, IB, CHUNK), jnp.int32),
          pltpu.VMEM((2, CHUNK, F), jnp.float32),
          pltpu.VMEM_SHARED((N_PAD, F), jnp.float32),
          pltpu.SemaphoreType.DMA,
          pltpu.SemaphoreType.DMA,
          pltpu.SemaphoreType.DMA,
      ],
  )
  def k(h_hbm, src0_hbm, dst0_hbm, src1_hbm, dst1_hbm, z_hbm, out_hbm,
        sidx, didx, rows, acc, gsem, isem, ssem):
    c = lax.axis_index("c")
    s = lax.axis_index("s")

    def pipeline(src_hbm, dst_hbm, nchunk):
      nb = nchunk // IB

      def block_descs(ib, slot):
        return (
            pltpu.make_async_copy(
                src_hbm.at[s, pl.ds(ib * IB, IB)], sidx.at[slot], isem),
            pltpu.make_async_copy(
                dst_hbm.at[s, pl.ds(ib * IB, IB)], didx.at[slot], isem),
        )

      def start_block(ib, slot):
        da, db = block_descs(ib, slot)
        da.start()
        db.start()

      def wait_block(ib, slot):
        da, db = block_descs(ib, slot)
        da.wait()
        db.wait()

      def gather(j, p):
        slot = lax.rem(j // IB, 3)
        jo = lax.rem(j, IB)
        return pltpu.make_async_copy(
            h_hbm.at[sidx.at[slot, jo]], rows.at[p], gsem)

      def scatter(j, p):
        slot = lax.rem(j // IB, 3)
        jo = lax.rem(j, IB)
        return pltpu.make_async_copy(
            rows.at[p], acc.at[didx.at[slot, jo]], ssem)

      start_block(0, 0)
      wait_block(0, 0)
      start_block(1, 1)
      gather(0, 0).start()

      def body(j, carry):
        p = lax.rem(j, 2)
        ib = j // IB
        jo = lax.rem(j, IB)
        gather(j, p).wait()

        @pl.when(j > 0)
        def _():
          scatter(j - 1, 1 - p).wait()

        @pl.when(jo == 0)
        def _():
          @pl.when(ib + 1 < nb)
          def _():
            wait_block(ib + 1, lax.rem(ib + 1, 3))

          @pl.when(ib + 2 < nb)
          def _():
            start_block(ib + 2, lax.rem(ib + 2, 3))

        @pl.when(j < nchunk - 1)
        def _():
          gather(j + 1, 1 - p).start()

        scatter(j, p).start(add=True)
        return carry

      lax.fori_loop(0, nchunk, body, 0)
      scatter(nchunk - 1, (nchunk - 1) % 2).wait()

    rows_per = N_PAD // NS
    pltpu.sync_copy(z_hbm.at[pl.ds(s * rows_per, rows_per)],
                    acc.at[pl.ds(s * rows_per, rows_per)])
    plsc.subcore_barrier()

    @pl.when(c == 0)
    def _():
      pipeline(src0_hbm, dst0_hbm, NCHUNK0)

    @pl.when(c == 1)
    def _():
      pipeline(src1_hbm, dst1_hbm, NCHUNK1)

    plsc.subcore_barrier()
    pltpu.sync_copy(acc.at[pl.ds(s * rows_per, rows_per)],
                    out_hbm.at[c, pl.ds(s * rows_per, rows_per)])

  return k(h, src0_t, dst0_t, src1_t, dst1_t, zeros)


def _dot(a, b):
  return jnp.dot(a, b, preferred_element_type=jnp.float32)


def _row_mask():
  i = pl.program_id(0)
  rows = i * BLK + lax.broadcasted_iota(jnp.int32, (BLK, 1), 0)
  return (rows < N).astype(jnp.float32)


def _conv_body(p_ref, h_ref, wrel_ref, wroot_ref, b_ref, o_ref):
  agg = p_ref[0] + p_ref[1]
  acc = _dot(agg, wrel_ref[...]) + _dot(h_ref[...], wroot_ref[...])
  o_ref[...] = jnp.maximum(acc + b_ref[...], 0.0) * _row_mask()


def _conv1(p, h, wrel, wroot, b):
  dout = wrel.shape[1]
  grid = (N_PAD // BLK,)
  return pl.pallas_call(
      _conv_body,
      grid=grid,
      in_specs=[
          pl.BlockSpec((NC, BLK, F), lambda i: (0, i, 0)),
          pl.BlockSpec((BLK, F), lambda i: (i, 0)),
          pl.BlockSpec(wrel.shape, lambda i: (0, 0)),
          pl.BlockSpec(wroot.shape, lambda i: (0, 0)),
          pl.BlockSpec(b.shape, lambda i: (0, 0)),
      ],
      out_specs=pl.BlockSpec((BLK, dout), lambda i: (i, 0)),
      out_shape=jax.ShapeDtypeStruct((N_PAD, dout), jnp.float32),
  )(p, h, wrel, wroot, b)


def _conv2_body(p_ref, h_ref, wrel_ref, wroot_ref, b_ref, oa_ref, ob_ref):
  agg = p_ref[0] + p_ref[1]
  acc = _dot(agg, wrel_ref[...]) + _dot(h_ref[...], wroot_ref[...])
  h2 = jnp.maximum(acc + b_ref[...], 0.0) * _row_mask()
  oa_ref[...] = h2[:, :F]
  ob_ref[...] = h2[:, F:]


def _conv2(p, h, wrel, wroot, b):
  grid = (N_PAD // BLK,)
  return pl.pallas_call(
      _conv2_body,
      grid=grid,
      in_specs=[
          pl.BlockSpec((NC, BLK, F), lambda i: (0, i, 0)),
          pl.BlockSpec((BLK, F), lambda i: (i, 0)),
          pl.BlockSpec(wrel.shape, lambda i: (0, 0)),
          pl.BlockSpec(wroot.shape, lambda i: (0, 0)),
          pl.BlockSpec(b.shape, lambda i: (0, 0)),
      ],
      out_specs=[
          pl.BlockSpec((BLK, F), lambda i: (i, 0)),
          pl.BlockSpec((BLK, F), lambda i: (i, 0)),
      ],
      out_shape=[
          jax.ShapeDtypeStruct((N_PAD, F), jnp.float32),
          jax.ShapeDtypeStruct((N_PAD, F), jnp.float32),
      ],
  )(p, h, wrel, wroot, b)


def _conv3_body(pa_ref, pb_ref, ha_ref, hb_ref, w3a_ref, w3b_ref, w3ra_ref,
                w3rb_ref, b_ref, w4rel_ref, o3_ref, o4_ref):
  agg_a = pa_ref[0] + pa_ref[1]
  agg_b = pb_ref[0] + pb_ref[1]
  acc = (_dot(agg_a, w3a_ref[...]) + _dot(agg_b, w3b_ref[...])
         + _dot(ha_ref[...], w3ra_ref[...]) + _dot(hb_ref[...], w3rb_ref[...]))
  h3 = jnp.maximum(acc + b_ref[...], 0.0)
  o3_ref[...] = h3
  o4_ref[...] = _dot(h3, w4rel_ref[...]) * _row_mask()


def _conv3(pa, pb, ha, hb, w3a, w3b, w3ra, w3rb, b, w4rel):
  grid = (N_PAD // BLK,)
  wspec = lambda w: pl.BlockSpec(w.shape, lambda i: (0, 0))
  return pl.pallas_call(
      _conv3_body,
      grid=grid,
      in_specs=[
          pl.BlockSpec((NC, BLK, F), lambda i: (0, i, 0)),
          pl.BlockSpec((NC, BLK, F), lambda i: (0, i, 0)),
          pl.BlockSpec((BLK, F), lambda i: (i, 0)),
          pl.BlockSpec((BLK, F), lambda i: (i, 0)),
          wspec(w3a), wspec(w3b), wspec(w3ra), wspec(w3rb), wspec(b),
          wspec(w4rel),
      ],
      out_specs=[
          pl.BlockSpec((BLK, 3 * F), lambda i: (i, 0)),
          pl.BlockSpec((BLK, F), lambda i: (i, 0)),
      ],
      out_shape=[
          jax.ShapeDtypeStruct((N_PAD, 3 * F), jnp.float32),
          jax.ShapeDtypeStruct((N_PAD, F), jnp.float32),
      ],
  )(pa, pb, ha, hb, w3a, w3b, w3ra, w3rb, b, w4rel)


def _pool_body(p_ref, h3_ref, batch_ref, w4root_ref, b4_ref, wh_ref, bh_ref,
               o_ref, sums, counts):
  i = pl.program_id(0)

  @pl.when(i == 0)
  def _():
    sums[...] = jnp.zeros_like(sums)
    counts[...] = jnp.zeros_like(counts)

  h4 = (p_ref[0] + p_ref[1] + _dot(h3_ref[...], w4root_ref[...])
        + b4_ref[...])
  seg = batch_ref[0]
  iota = lax.broadcasted_iota(jnp.int32, (G, BLK), 0)
  onehot_t = (iota == seg).astype(jnp.float32)
  sums[...] += _dot(onehot_t, h4)
  cnt = jnp.sum(onehot_t, axis=1, keepdims=True)
  counts[...] += jnp.broadcast_to(cnt, counts.shape)

  @pl.when(i == pl.num_programs(0) - 1)
  def _():
    pooled = sums[...] / jnp.maximum(counts[...], 1.0)
    o_ref[...] = _dot(pooled, wh_ref[...]) + bh_ref[...]


def _pool(p, h3, batch3, w4root, b4, wh_pad, bh_b):
  grid = (N_PAD // BLK,)
  wspec = lambda w: pl.BlockSpec(w.shape, lambda i: (0, 0))
  return pl.pallas_call(
      _pool_body,
      grid=grid,
      in_specs=[
          pl.BlockSpec((NC, BLK, F), lambda i: (0, i, 0)),
          pl.BlockSpec((BLK, 3 * F), lambda i: (i, 0)),
          pl.BlockSpec((1, 1, BLK), lambda i: (i, 0, 0)),
          wspec(w4root), wspec(b4), wspec(wh_pad), wspec(bh_b),
      ],
      out_specs=pl.BlockSpec((G, F), lambda i: (0, 0)),
      out_shape=jax.ShapeDtypeStruct((G, F), jnp.float32),
      scratch_shapes=[
          pltpu.VMEM((G, F), jnp.float32),
          pltpu.VMEM((G, F), jnp.float32),
      ],
  )(p, h3, batch3, w4root, b4, wh_pad, bh_b)


def kernel(x, edge_index, batch, W1_rel, b1, W1_root, W2_rel, b2, W2_root,
           W3_rel, b3, W3_root, W4_rel, b4, W4_root, Wh, bh):
  src, dst = edge_index[0], edge_index[1]
  pad_e = E_PAD - E
  ar = jnp.arange(pad_e, dtype=jnp.int32)
  pad_src = N + 8 + ar % (N_PAD - N - 16)
  pad_dst = (ar * 17) % N_PAD
  src_p = jnp.concatenate([src, pad_src])
  dst_p = jnp.concatenate([dst, pad_dst])
  src0_t = src_p[:E_SPLIT].reshape(NS, NCHUNK0, CHUNK)
  dst0_t = dst_p[:E_SPLIT].reshape(NS, NCHUNK0, CHUNK)
  src1_t = src_p[E_SPLIT:].reshape(NS, NCHUNK1, CHUNK)
  dst1_t = dst_p[E_SPLIT:].reshape(NS, NCHUNK1, CHUNK)
  zeros = jnp.zeros((N_PAD, F), jnp.float32)

  x_pad = jnp.pad(x, ((0, N_PAD - N), (0, 0)))
  batch3 = jnp.pad(batch, (0, N_PAD - N), constant_values=G).reshape(
      N_PAD // BLK, 1, BLK)

  b1r, b2r, b3r, b4r = (b.reshape(1, -1) for b in (b1, b2, b3, b4))
  w3a, w3b = W3_rel[:F], W3_rel[F:]
  w3ra, w3rb = W3_root[:F], W3_root[F:]
  wh_pad = jnp.pad(Wh, ((0, 0), (0, F - Wh.shape[1])))
  bh_b = jnp.broadcast_to(bh.reshape(1, -1), (1, F))

  p1 = _seg_sum_sc(x_pad, src0_t, dst0_t, src1_t, dst1_t, zeros)
  h1 = _conv1(p1, x_pad, W1_rel, W1_root, b1r)
  p2 = _seg_sum_sc(h1, src0_t, dst0_t, src1_t, dst1_t, zeros)
  h2a, h2b = _conv2(p2, h1, W2_rel, W2_root, b2r)
  p3a = _seg_sum_sc(h2a, src0_t, dst0_t, src1_t, dst1_t, zeros)
  p3b = _seg_sum_sc(h2b, src0_t, dst0_t, src1_t, dst1_t, zeros)
  h3, y4 = _conv3(p3a, p3b, h2a, h2b, w3a, w3b, w3ra, w3rb, b3r, W4_rel)
  p4 = _seg_sum_sc(y4, src0_t, dst0_t, src1_t, dst1_t, zeros)
  out_full = _pool(p4, h3, batch3, W4_root, b4r, wh_pad, bh_b)
  return out_full[:, :1]
